# baseline (speedup 1.0000x reference)
import functools
import jax
import jax.numpy as jnp
from jax import lax
from jax.experimental import pallas as pl
from jax.experimental.pallas import tpu as pltpu
import jax.experimental.pallas.tpu_sc as plsc

_NUM_ACTIONS = 1000
_S = 256
_ROW = _S + _NUM_ACTIONS

_NC = 2
_NS = 16
_L = 16
_NW = _NC * _NS


def _dense_t_body(flat_t_ref, out_ref):
    C = out_ref.shape[1]
    out_ref[:_S, :] = flat_t_ref[...]
    out_ref[_S:, :] = jnp.zeros((_NUM_ACTIONS, C), jnp.float32)


def _dense_t(flat_t, B):
    C = 2048
    return pl.pallas_call(
        _dense_t_body,
        grid=(B // C,),
        in_specs=[pl.BlockSpec((_S, C), lambda i: (0, i))],
        out_specs=pl.BlockSpec((_ROW, C), lambda i: (0, i)),
        out_shape=jax.ShapeDtypeStruct((_ROW, B), jnp.float32),
        compiler_params=pltpu.CompilerParams(dimension_semantics=("arbitrary",)),
    )(flat_t)


def _make_sc_scatter(B):
    per_w = B // _NW
    n_chunk = per_w // 128
    mesh = plsc.VectorSubcoreMesh(core_axis_name="c", subcore_axis_name="s")

    @functools.partial(
        pl.kernel,
        out_type=(),
        mesh=mesh,
        scratch_types=[
            pltpu.VMEM((per_w,), jnp.int32),
            pltpu.VMEM((n_chunk, 128), jnp.int32),
            pltpu.VMEM((128,), jnp.float32),
            pltpu.SemaphoreType.DMA,
            pltpu.SemaphoreType.DMA,
        ],
    )
    def sc_scatter(actions_hbm, out1d_ref, act_v, idx_v, ones_v, asem, ssem):
        wid = lax.axis_index("s") * _NC + lax.axis_index("c")
        base = wid * per_w
        acopy = pltpu.make_async_copy(
            actions_hbm.at[pl.ds(base, per_w)], act_v, asem
        )
        acopy.start()
        for j in range(128 // _L):
            ones_v[pl.ds(j * _L, _L)] = jnp.full((_L,), 1.0, jnp.float32)
        acopy.wait()
        handles = []
        for rr in range(n_chunk):
            for jj in range(128 // _L):
                j = rr * (128 // _L) + jj
                a = act_v[pl.ds(j * _L, _L)]
                b = (base + j * _L) + lax.broadcasted_iota(jnp.int32, (_L,), 0)
                r = a + _S
                p = (
                    ((((r >> 3) << 7) | (b >> 7)) << 10)
                    | ((r & 7) << 7)
                    | (b & 127)
                )
                idx_v[rr, pl.ds(jj * _L, _L)] = p
            handles.append(
                pltpu.async_copy(ones_v, out1d_ref.at[idx_v.at[rr]], ssem)
            )
        for h in handles:
            h.wait()

    return sc_scatter


def kernel(states, actions):
    B = states.shape[0]
    flat_t = states.reshape(B, _S).T
    dense_t = _dense_t(flat_t, B)
    view1d = (
        dense_t.reshape(_ROW // 8, 8, B // 128, 128)
        .transpose(0, 2, 1, 3)
        .reshape(_ROW * B)
    )
    out_ref = jax.new_ref(view1d)
    _make_sc_scatter(B)(actions.astype(jnp.int32), out_ref)
    res = out_ref[...]
    out_t = (
        res.reshape(_ROW // 8, B // 128, 8, 128)
        .transpose(0, 2, 1, 3)
        .reshape(_ROW, B)
    )
    return out_t.T


# speedup vs baseline: 1.0021x; 1.0021x over previous
import functools
import jax
import jax.numpy as jnp
from jax import lax
from jax.experimental import pallas as pl
from jax.experimental.pallas import tpu as pltpu
import jax.experimental.pallas.tpu_sc as plsc

_NUM_ACTIONS = 1000
_S = 256
_ROW = _S + _NUM_ACTIONS

_NC = 2
_NS = 16
_L = 16
_NW = _NC * _NS


def _dense_t_body(flat_t_ref, out_ref):
    C = out_ref.shape[1]
    out_ref[:_S, :] = flat_t_ref[...]
    out_ref[_S:, :] = jnp.zeros((_NUM_ACTIONS, C), jnp.float32)


def _dense_t(flat_t, B):
    C = 2048
    return pl.pallas_call(
        _dense_t_body,
        grid=(B // C,),
        in_specs=[pl.BlockSpec((_S, C), lambda i: (0, i))],
        out_specs=pl.BlockSpec((_ROW, C), lambda i: (0, i)),
        out_shape=jax.ShapeDtypeStruct((_ROW, B), jnp.float32),
        compiler_params=pltpu.CompilerParams(dimension_semantics=("arbitrary",)),
    )(flat_t)


def _make_sc_scatter(B):
    per_w = B // _NW
    n_chunk = per_w // 128
    mesh = plsc.VectorSubcoreMesh(core_axis_name="c", subcore_axis_name="s")

    @functools.partial(
        pl.kernel,
        out_type=(),
        mesh=mesh,
        scratch_types=[
            pltpu.VMEM((per_w,), jnp.int32),
            pltpu.VMEM((per_w,), jnp.int32),
            pltpu.VMEM((per_w,), jnp.float32),
            pltpu.SemaphoreType.DMA,
            pltpu.SemaphoreType.DMA,
        ],
    )
    def sc_scatter(actions_hbm, out1d_ref, act_v, idx_v, ones_v, asem, ssem):
        wid = lax.axis_index("s") * _NC + lax.axis_index("c")
        base = wid * per_w
        acopy = pltpu.make_async_copy(
            actions_hbm.at[pl.ds(base, per_w)], act_v, asem
        )
        acopy.start()
        for j in range(per_w // _L):
            ones_v[pl.ds(j * _L, _L)] = jnp.full((_L,), 1.0, jnp.float32)
        acopy.wait()
        for j in range(per_w // _L):
            a = act_v[pl.ds(j * _L, _L)]
            b = (base + j * _L) + lax.broadcasted_iota(jnp.int32, (_L,), 0)
            r = a + _S
            p = (
                ((((r >> 3) << 7) | (b >> 7)) << 10)
                | ((r & 7) << 7)
                | (b & 127)
            )
            idx_v[pl.ds(j * _L, _L)] = p
        pltpu.async_copy(ones_v, out1d_ref.at[idx_v], ssem).wait()

    return sc_scatter


def kernel(states, actions):
    B = states.shape[0]
    flat_t = states.reshape(B, _S).T
    dense_t = _dense_t(flat_t, B)
    view1d = (
        dense_t.reshape(_ROW // 8, 8, B // 128, 128)
        .transpose(0, 2, 1, 3)
        .reshape(_ROW * B)
    )
    out_ref = jax.new_ref(view1d)
    _make_sc_scatter(B)(actions.astype(jnp.int32), out_ref)
    res = out_ref[...]
    out_t = (
        res.reshape(_ROW // 8, B // 128, 8, 128)
        .transpose(0, 2, 1, 3)
        .reshape(_ROW, B)
    )
    return out_t.T


# speedup vs baseline: 1.0145x; 1.0124x over previous
import functools
import jax
import jax.numpy as jnp
from jax import lax
from jax.experimental import pallas as pl
from jax.experimental.pallas import tpu as pltpu
import jax.experimental.pallas.tpu_sc as plsc

_NUM_ACTIONS = 1000
_S = 256
_ROW = _S + _NUM_ACTIONS

_NC = 2
_NS = 16
_L = 16
_NW = _NC * _NS


def _dense_t_body(flat_t_ref, out_ref):
    C = out_ref.shape[1]
    out_ref[:_S, :] = flat_t_ref[...]
    out_ref[_S:, :] = jnp.zeros((_NUM_ACTIONS, C), jnp.float32)


def _dense_t(flat_t, B):
    C = 4096
    return pl.pallas_call(
        _dense_t_body,
        grid=(B // C,),
        in_specs=[pl.BlockSpec((_S, C), lambda i: (0, i))],
        out_specs=pl.BlockSpec((_ROW, C), lambda i: (0, i)),
        out_shape=jax.ShapeDtypeStruct((_ROW, B), jnp.float32),
        compiler_params=pltpu.CompilerParams(dimension_semantics=("arbitrary",)),
    )(flat_t)


def _make_sc_scatter(B):
    per_w = B // _NW
    n_chunk = per_w // 128
    mesh = plsc.VectorSubcoreMesh(core_axis_name="c", subcore_axis_name="s")

    @functools.partial(
        pl.kernel,
        out_type=(),
        mesh=mesh,
        scratch_types=[
            pltpu.VMEM((per_w,), jnp.int32),
            pltpu.VMEM((per_w,), jnp.int32),
            pltpu.VMEM((per_w,), jnp.float32),
            pltpu.SemaphoreType.DMA,
            pltpu.SemaphoreType.DMA,
        ],
    )
    def sc_scatter(actions_hbm, out1d_ref, act_v, idx_v, ones_v, asem, ssem):
        wid = lax.axis_index("s") * _NC + lax.axis_index("c")
        base = wid * per_w
        acopy = pltpu.make_async_copy(
            actions_hbm.at[pl.ds(base, per_w)], act_v, asem
        )
        acopy.start()
        for j in range(per_w // _L):
            ones_v[pl.ds(j * _L, _L)] = jnp.full((_L,), 1.0, jnp.float32)
        acopy.wait()
        for j in range(per_w // _L):
            a = act_v[pl.ds(j * _L, _L)]
            b = (base + j * _L) + lax.broadcasted_iota(jnp.int32, (_L,), 0)
            r = a + _S
            p = (
                ((((r >> 3) << 7) | (b >> 7)) << 10)
                | ((r & 7) << 7)
                | (b & 127)
            )
            idx_v[pl.ds(j * _L, _L)] = p
        pltpu.async_copy(ones_v, out1d_ref.at[idx_v], ssem).wait()

    return sc_scatter


def kernel(states, actions):
    B = states.shape[0]
    flat_t = states.reshape(B, _S).T
    dense_t = _dense_t(flat_t, B)
    view1d = (
        dense_t.reshape(_ROW // 8, 8, B // 128, 128)
        .transpose(0, 2, 1, 3)
        .reshape(_ROW * B)
    )
    out_ref = jax.new_ref(view1d)
    _make_sc_scatter(B)(actions.astype(jnp.int32), out_ref)
    res = out_ref[...]
    out_t = (
        res.reshape(_ROW // 8, B // 128, 8, 128)
        .transpose(0, 2, 1, 3)
        .reshape(_ROW, B)
    )
    return out_t.T
